# baseline (speedup 1.0000x reference)
import functools

import jax
import jax.numpy as jnp
from jax import lax
from jax.experimental import pallas as pl
from jax.experimental.pallas import tpu as pltpu
from jax.experimental.pallas import tpu_sc as plsc

VOCAB_SIZE = 1000000
EMBED_DIM = 64
BATCH = 16384
HIST_LEN = 200

NC = 2
NS = 16
NW = NC * NS

TOTAL = BATCH * HIST_LEN
IDX_W = 128
N_IDX_ROWS = TOTAL // IDX_W
ROWS_PER_W = N_IDX_ROWS // NW
K = 4
N_CHUNKS = ROWS_PER_W // K
CHUNK_ROWS = K * IDX_W


def _sc_gather(idx2d, table):
    mesh = plsc.VectorSubcoreMesh(
        core_axis_name="c", subcore_axis_name="s",
        num_cores=NC, num_subcores=NS,
    )

    @functools.partial(
        pl.kernel,
        out_type=jax.ShapeDtypeStruct((TOTAL, EMBED_DIM), jnp.float32),
        mesh=mesh,
        scratch_types=[
            pltpu.VMEM((K, IDX_W), jnp.int32),
            pltpu.VMEM((CHUNK_ROWS, EMBED_DIM), jnp.float32),
            pltpu.SemaphoreType.DMA,
        ],
        compiler_params=pltpu.CompilerParams(use_tc_tiling_on_sc=False),
    )
    def body(idx_hbm, table_hbm, out_hbm, idx_v, rows_v, sem):
        wid = lax.axis_index("s") * NC + lax.axis_index("c")
        row0 = wid * ROWS_PER_W

        @pl.loop(0, N_CHUNKS)
        def _chunk(g):
            base = row0 + g * K
            pltpu.sync_copy(idx_hbm.at[pl.ds(base, K)], idx_v)
            descs = [
                pltpu.async_copy(
                    table_hbm.at[idx_v.at[j]],
                    rows_v.at[pl.ds(j * IDX_W, IDX_W)],
                    sem,
                )
                for j in range(K)
            ]
            for d in descs:
                d.wait()
            pltpu.sync_copy(rows_v, out_hbm.at[pl.ds(base * IDX_W, CHUNK_ROWS)])

    return body(idx2d, table)


def kernel(inputs, table):
    idx2d = inputs.reshape(N_IDX_ROWS, IDX_W).astype(jnp.int32)
    out = _sc_gather(idx2d, table)
    return out.reshape(BATCH, HIST_LEN, EMBED_DIM)


# speedup vs baseline: 1.0763x; 1.0763x over previous
import functools

import jax
import jax.numpy as jnp
from jax import lax
from jax.experimental import pallas as pl
from jax.experimental.pallas import tpu as pltpu
from jax.experimental.pallas import tpu_sc as plsc

VOCAB_SIZE = 1000000
EMBED_DIM = 64
BATCH = 16384
HIST_LEN = 200

NC = 2
NS = 16
NW = NC * NS

TOTAL = BATCH * HIST_LEN
IDX_W = 128
N_IDX_ROWS = TOTAL // IDX_W
ROWS_PER_W = N_IDX_ROWS // NW
K = 4
N_CHUNKS = ROWS_PER_W // K
CHUNK_ROWS = K * IDX_W


def _sc_gather(idx2d, table):
    mesh = plsc.VectorSubcoreMesh(
        core_axis_name="c", subcore_axis_name="s",
        num_cores=NC, num_subcores=NS,
    )

    @functools.partial(
        pl.kernel,
        out_type=jax.ShapeDtypeStruct((TOTAL, EMBED_DIM), jnp.float32),
        mesh=mesh,
        scratch_types=[
            pltpu.VMEM((K, IDX_W), jnp.int32),
            pltpu.VMEM((K, IDX_W), jnp.int32),
            pltpu.VMEM((CHUNK_ROWS, EMBED_DIM), jnp.float32),
            pltpu.VMEM((CHUNK_ROWS, EMBED_DIM), jnp.float32),
            pltpu.SemaphoreType.DMA,
            pltpu.SemaphoreType.DMA,
            pltpu.SemaphoreType.DMA,
            pltpu.SemaphoreType.DMA,
            pltpu.SemaphoreType.DMA,
            pltpu.SemaphoreType.DMA,
        ],
        compiler_params=pltpu.CompilerParams(use_tc_tiling_on_sc=False),
    )
    def body(idx_hbm, table_hbm, out_hbm, idx0, idx1, rows0, rows1,
             gat0, gat1, wr0, wr1, ix0, ix1):
        idx_v = (idx0, idx1)
        rows_v = (rows0, rows1)
        gat = (gat0, gat1)
        wr = (wr0, wr1)
        ixs = (ix0, ix1)

        wid = lax.axis_index("s") * NC + lax.axis_index("c")
        row0 = wid * ROWS_PER_W

        def fire_gather(b, g):
            for j in range(K):
                pltpu.async_copy(
                    table_hbm.at[idx_v[b].at[j]],
                    rows_v[b].at[pl.ds(j * IDX_W, IDX_W)],
                    gat[b],
                )

        def drain_gather(b):
            pltpu.make_async_copy(
                table_hbm.at[pl.ds(0, CHUNK_ROWS)], rows_v[b], gat[b]
            ).wait()

        def fire_writeback(b, g):
            base = (row0 + g * K) * IDX_W
            pltpu.async_copy(
                rows_v[b], out_hbm.at[pl.ds(base, CHUNK_ROWS)], wr[b]
            )

        def drain_writeback(b):
            pltpu.make_async_copy(
                rows_v[b], out_hbm.at[pl.ds(0, CHUNK_ROWS)], wr[b]
            ).wait()

        def fire_idx(b, g):
            pltpu.async_copy(
                idx_hbm.at[pl.ds(row0 + g * K, K)], idx_v[b], ixs[b]
            )

        def drain_idx(b):
            pltpu.make_async_copy(
                idx_hbm.at[pl.ds(0, K)], idx_v[b], ixs[b]
            ).wait()

        pltpu.sync_copy(idx_hbm.at[pl.ds(row0, K)], idx0)
        fire_idx(1, 1)
        fire_gather(0, 0)

        @pl.loop(0, N_CHUNKS, step=2)
        def _pair(g0):
            for b in range(2):
                g = g0 + b
                nb = 1 - b
                drain_gather(b)
                fire_writeback(b, g)
                @pl.when(g + 2 < N_CHUNKS)
                def _():
                    fire_idx(b, g + 2)
                @pl.when(g > 0)
                def _():
                    drain_writeback(nb)
                @pl.when(g + 1 < N_CHUNKS)
                def _():
                    drain_idx(nb)
                    fire_gather(nb, g + 1)

        drain_writeback(1)

    return body(idx2d, table)


def kernel(inputs, table):
    idx2d = inputs.reshape(N_IDX_ROWS, IDX_W).astype(jnp.int32)
    out = _sc_gather(idx2d, table)
    return out.reshape(BATCH, HIST_LEN, EMBED_DIM)


# speedup vs baseline: 1.0787x; 1.0022x over previous
import functools

import jax
import jax.numpy as jnp
from jax import lax
from jax.experimental import pallas as pl
from jax.experimental.pallas import tpu as pltpu
from jax.experimental.pallas import tpu_sc as plsc

VOCAB_SIZE = 1000000
EMBED_DIM = 64
BATCH = 16384
HIST_LEN = 200

NC = 2
NS = 16
NW = NC * NS

B_PER_W = BATCH // NW
CB = 4
N_CHUNKS = B_PER_W // CB
SPLITS = ((0, 128), (128, 72))


def _sc_gather(idx, table):
    mesh = plsc.VectorSubcoreMesh(
        core_axis_name="c", subcore_axis_name="s",
        num_cores=NC, num_subcores=NS,
    )

    @functools.partial(
        pl.kernel,
        out_type=jax.ShapeDtypeStruct((BATCH, HIST_LEN, EMBED_DIM), jnp.float32),
        mesh=mesh,
        scratch_types=[
            pltpu.VMEM((CB, HIST_LEN), jnp.int32),
            pltpu.VMEM((CB, HIST_LEN), jnp.int32),
            pltpu.VMEM((CB, HIST_LEN, EMBED_DIM), jnp.float32),
            pltpu.VMEM((CB, HIST_LEN, EMBED_DIM), jnp.float32),
            pltpu.SemaphoreType.DMA,
            pltpu.SemaphoreType.DMA,
            pltpu.SemaphoreType.DMA,
            pltpu.SemaphoreType.DMA,
            pltpu.SemaphoreType.DMA,
            pltpu.SemaphoreType.DMA,
        ],
        compiler_params=pltpu.CompilerParams(use_tc_tiling_on_sc=False),
    )
    def body(idx_hbm, table_hbm, out_hbm, idx0, idx1, rows0, rows1,
             gat0, gat1, wr0, wr1, ix0, ix1):
        idx_v = (idx0, idx1)
        rows_v = (rows0, rows1)
        gat = (gat0, gat1)
        wr = (wr0, wr1)
        ixs = (ix0, ix1)

        wid = lax.axis_index("s") * NC + lax.axis_index("c")
        b_start = wid * B_PER_W

        def fire_gather(b, g):
            for i in range(CB):
                for off, width in SPLITS:
                    pltpu.async_copy(
                        table_hbm.at[idx_v[b].at[i, pl.ds(off, width)]],
                        rows_v[b].at[i, pl.ds(off, width)],
                        gat[b],
                    )

        def drain_gather(b):
            pltpu.make_async_copy(
                out_hbm.at[pl.ds(0, CB)], rows_v[b], gat[b]
            ).wait()

        def fire_writeback(b, g):
            pltpu.async_copy(
                rows_v[b], out_hbm.at[pl.ds(b_start + g * CB, CB)], wr[b]
            )

        def drain_writeback(b):
            pltpu.make_async_copy(
                rows_v[b], out_hbm.at[pl.ds(0, CB)], wr[b]
            ).wait()

        def fire_idx(b, g):
            pltpu.async_copy(
                idx_hbm.at[pl.ds(b_start + g * CB, CB)], idx_v[b], ixs[b]
            )

        def drain_idx(b):
            pltpu.make_async_copy(
                idx_hbm.at[pl.ds(0, CB)], idx_v[b], ixs[b]
            ).wait()

        pltpu.sync_copy(idx_hbm.at[pl.ds(b_start, CB)], idx0)
        fire_idx(1, 1)
        fire_gather(0, 0)

        @pl.loop(0, N_CHUNKS, step=2)
        def _pair(g0):
            for b in range(2):
                g = g0 + b
                nb = 1 - b
                drain_gather(b)
                fire_writeback(b, g)
                @pl.when(g + 2 < N_CHUNKS)
                def _():
                    fire_idx(b, g + 2)
                @pl.when(g > 0)
                def _():
                    drain_writeback(nb)
                @pl.when(g + 1 < N_CHUNKS)
                def _():
                    drain_idx(nb)
                    fire_gather(nb, g + 1)

        drain_writeback(1)

    return body(idx, table)


def kernel(inputs, table):
    return _sc_gather(inputs.astype(jnp.int32), table)


# speedup vs baseline: 1.5414x; 1.4290x over previous
import functools

import jax
import jax.numpy as jnp
from jax import lax
from jax.experimental import pallas as pl
from jax.experimental.layout import Layout, with_layout_constraint
from jax.experimental.pallas import tpu as pltpu
from jax.experimental.pallas import tpu_sc as plsc

VOCAB_SIZE = 1000000
EMBED_DIM = 64
BATCH = 16384
HIST_LEN = 200

NC = 2
NS = 16
NW = NC * NS

B_PER_W = BATCH // NW
CB = 4
N_CHUNKS = B_PER_W // CB
SPLITS = ((0, 128), (128, 72))


def _sc_gather(idx, table):
    mesh = plsc.VectorSubcoreMesh(
        core_axis_name="c", subcore_axis_name="s",
        num_cores=NC, num_subcores=NS,
    )

    @functools.partial(
        pl.kernel,
        out_type=jax.ShapeDtypeStruct((BATCH, HIST_LEN, EMBED_DIM), jnp.float32),
        mesh=mesh,
        scratch_types=[
            pltpu.VMEM((CB, HIST_LEN), jnp.int32),
            pltpu.VMEM((CB, HIST_LEN), jnp.int32),
            pltpu.VMEM((CB, HIST_LEN, EMBED_DIM), jnp.float32),
            pltpu.VMEM((CB, HIST_LEN, EMBED_DIM), jnp.float32),
            pltpu.SemaphoreType.DMA,
            pltpu.SemaphoreType.DMA,
            pltpu.SemaphoreType.DMA,
            pltpu.SemaphoreType.DMA,
            pltpu.SemaphoreType.DMA,
            pltpu.SemaphoreType.DMA,
        ],
        compiler_params=pltpu.CompilerParams(use_tc_tiling_on_sc=False),
    )
    def body(idx_hbm, table_hbm, out_hbm, idx0, idx1, rows0, rows1,
             gat0, gat1, wr0, wr1, ix0, ix1):
        idx_v = (idx0, idx1)
        rows_v = (rows0, rows1)
        gat = (gat0, gat1)
        wr = (wr0, wr1)
        ixs = (ix0, ix1)

        wid = lax.axis_index("s") * NC + lax.axis_index("c")
        b_start = wid * B_PER_W

        def fire_gather(b, g):
            for i in range(CB):
                for off, width in SPLITS:
                    pltpu.async_copy(
                        table_hbm.at[idx_v[b].at[i, pl.ds(off, width)]],
                        rows_v[b].at[i, pl.ds(off, width)],
                        gat[b],
                    )

        def drain_gather(b):
            pltpu.make_async_copy(
                out_hbm.at[pl.ds(0, CB)], rows_v[b], gat[b]
            ).wait()

        def fire_writeback(b, g):
            pltpu.async_copy(
                rows_v[b], out_hbm.at[pl.ds(b_start + g * CB, CB)], wr[b]
            )

        def drain_writeback(b):
            pltpu.make_async_copy(
                rows_v[b], out_hbm.at[pl.ds(0, CB)], wr[b]
            ).wait()

        def fire_idx(b, g):
            pltpu.async_copy(
                idx_hbm.at[pl.ds(b_start + g * CB, CB)], idx_v[b], ixs[b]
            )

        def drain_idx(b):
            pltpu.make_async_copy(
                idx_hbm.at[pl.ds(0, CB)], idx_v[b], ixs[b]
            ).wait()

        pltpu.sync_copy(idx_hbm.at[pl.ds(b_start, CB)], idx0)
        fire_idx(1, 1)
        fire_gather(0, 0)

        @pl.loop(0, N_CHUNKS, step=2)
        def _pair(g0):
            for b in range(2):
                g = g0 + b
                nb = 1 - b
                drain_gather(b)
                fire_writeback(b, g)
                @pl.when(g + 2 < N_CHUNKS)
                def _():
                    fire_idx(b, g + 2)
                @pl.when(g > 0)
                def _():
                    drain_writeback(nb)
                @pl.when(g + 1 < N_CHUNKS)
                def _():
                    drain_idx(nb)
                    fire_gather(nb, g + 1)

        drain_writeback(1)

    return body(idx, table)


def kernel(inputs, table):
    table = with_layout_constraint(
        table, Layout(major_to_minor=(0, 1), tiling=()))
    out = _sc_gather(inputs.astype(jnp.int32), table)
    return with_layout_constraint(out, Layout(major_to_minor=(0, 1, 2)))
